# Initial kernel scaffold; baseline (speedup 1.0000x reference)
import jax
import jax.numpy as jnp
from jax.experimental import pallas as pl


def kernel(vp, vt, x0, x1, segment_ids):
    raise NotImplementedError("write your pallas kernel here")


# speedup vs baseline: 3.4516x; 3.4516x over previous
import functools

import jax
import jax.numpy as jnp
from jax import lax
from jax.experimental import pallas as pl
from jax.experimental.pallas import tpu as pltpu
from jax.experimental.pallas import tpu_sc as plsc

N = 320000
D = 128
NSEG = 10000
NW = 32
CHUNK = N // NW
BT = 2560
RT = BT // 128
GRID = N // BT

_CSF = 0.5
_W = 0.1


def _tok_body(vp_ref, vt_ref, x0_ref, x1_ref, d_ref, c_ref, col_ref, r_ref):
    i = pl.program_id(0)
    vp = vp_ref[...]
    vt = vt_ref[...]
    diff = vp - vt
    diff2 = diff * diff
    d_row = jnp.sum(diff2, axis=1)
    num = jnp.sum(vp * vt, axis=1)
    na = jnp.sum(vp * vp, axis=1)
    nb = jnp.sum(vt * vt, axis=1)
    r = num / (jnp.sqrt(na) * jnp.sqrt(nb))
    x0 = x0_ref[...]
    x1 = x1_ref[...]
    cd = x0 - x1
    c_row = jnp.sqrt(jnp.sum(cd * cd, axis=1))

    d_ref[...] = d_row.reshape(1, RT, 128)
    c_ref[...] = c_row.reshape(1, RT, 128)

    col_part = jnp.sum(diff2, axis=0, keepdims=True)
    r_part = jnp.sum(r.reshape(RT, 128), axis=0, keepdims=True)

    @pl.when(i == 0)
    def _init():
        col_ref[...] = col_part
        r_ref[...] = r_part

    @pl.when(i > 0)
    def _acc():
        col_ref[...] = col_ref[...] + col_part
        r_ref[...] = r_ref[...] + r_part


_tok_call = pl.pallas_call(
    _tok_body,
    grid=(GRID,),
    in_specs=[pl.BlockSpec((BT, 128), lambda i: (i, 0))] * 4,
    out_specs=[
        pl.BlockSpec((1, RT, 128), lambda i: (i, 0, 0)),
        pl.BlockSpec((1, RT, 128), lambda i: (i, 0, 0)),
        pl.BlockSpec((1, 128), lambda i: (0, 0)),
        pl.BlockSpec((1, 128), lambda i: (0, 0)),
    ],
    out_shape=[
        jax.ShapeDtypeStruct((GRID, RT, 128), jnp.float32),
        jax.ShapeDtypeStruct((GRID, RT, 128), jnp.float32),
        jax.ShapeDtypeStruct((1, 128), jnp.float32),
        jax.ShapeDtypeStruct((1, 128), jnp.float32),
    ],
    compiler_params=pltpu.CompilerParams(
        dimension_semantics=("arbitrary",),
    ),
)


@functools.lru_cache(maxsize=1)
def _make_seg_call():
    mesh = plsc.VectorSubcoreMesh(core_axis_name="c", subcore_axis_name="s")

    @functools.partial(
        pl.kernel,
        mesh=mesh,
        out_type=[
            jax.ShapeDtypeStruct((NW, NSEG), jnp.float32),
            jax.ShapeDtypeStruct((NW, NSEG), jnp.float32),
            jax.ShapeDtypeStruct((NW, NSEG), jnp.float32),
        ],
        compiler_params=pltpu.CompilerParams(needs_layout_passes=False),
        scratch_types=[
            pltpu.VMEM((CHUNK + 16,), jnp.int32),
            pltpu.VMEM((CHUNK,), jnp.float32),
            pltpu.VMEM((CHUNK,), jnp.float32),
            pltpu.VMEM((NSEG,), jnp.float32),
            pltpu.VMEM((NSEG,), jnp.float32),
            pltpu.VMEM((NSEG,), jnp.float32),
        ],
    )
    def seg_kernel(ids_hbm, d_hbm, c_hbm, sd_out, sc_out, cnt_out,
                   ids_v, d_v, c_v, acc_d, acc_c, acc_n):
        w = lax.axis_index("s") * 2 + lax.axis_index("c")
        base = w * CHUNK
        pltpu.sync_copy(ids_hbm.at[pl.ds(base, CHUNK)],
                        ids_v.at[pl.ds(8, CHUNK)])
        pltpu.sync_copy(d_hbm.at[pl.ds(base, CHUNK)], d_v)
        pltpu.sync_copy(c_hbm.at[pl.ds(base, CHUNK)], c_v)

        zf = jnp.zeros((16,), jnp.float32)

        def zero_body(j, carry):
            acc_d[pl.ds(j * 16, 16)] = zf
            acc_c[pl.ds(j * 16, 16)] = zf
            acc_n[pl.ds(j * 16, 16)] = zf
            return carry

        lax.fori_loop(0, NSEG // 16, zero_body, 0)

        lane = lax.iota(jnp.int32, 16)
        lanef = lane.astype(jnp.float32)

        def body(j, carry):
            off = 8 + j * 16
            ids = ids_v[pl.ds(off, 16)]
            prv = ids_v[pl.ds(off - 1, 16)]
            nxt = ids_v[pl.ds(off + 1, 16)]
            start = (ids != prv) | (lane == 0)
            end = (ids != nxt) | (lane == 15)
            dv = d_v[pl.ds(j * 16, 16)]
            cv = c_v[pl.ds(j * 16, 16)]
            cs_d = plsc.cumsum(dv)
            cs_c = plsc.cumsum(cv)
            plsc.addupdate_scatter(acc_d, [ids], cs_d, mask=end)
            plsc.addupdate_scatter(acc_d, [ids], dv - cs_d, mask=start)
            plsc.addupdate_scatter(acc_c, [ids], cs_c, mask=end)
            plsc.addupdate_scatter(acc_c, [ids], cv - cs_c, mask=start)
            plsc.addupdate_scatter(acc_n, [ids], lanef + 1.0, mask=end)
            plsc.addupdate_scatter(acc_n, [ids], -lanef, mask=start)
            return carry

        lax.fori_loop(0, CHUNK // 16, body, 0)

        pltpu.sync_copy(acc_d, sd_out.at[w])
        pltpu.sync_copy(acc_c, sc_out.at[w])
        pltpu.sync_copy(acc_n, cnt_out.at[w])

    return seg_kernel


def _fin_body(sd_ref, sc_ref, cnt_ref, col_ref, r_ref, loss_ref, dp_ref):
    sd = jnp.sum(sd_ref[...], axis=0)
    sc = jnp.sum(sc_ref[...], axis=0)
    cnt = jnp.sum(cnt_ref[...], axis=0)
    safe = jnp.maximum(cnt, 1.0)
    rs = jnp.sum(r_ref[...])
    cs = 1.0 - rs / N
    loss_seg = ((1.0 - _CSF) * 0.5 * sd + _CSF * cs * cnt) / safe
    loss_seg = loss_seg * jnp.exp(-_W * sc / safe)
    loss_ref[...] = (jnp.sum(loss_seg) / NSEG).reshape(1, 1)
    dp_ref[...] = col_ref[...] * (0.5 / N)


_fin_call = pl.pallas_call(
    _fin_body,
    out_shape=[
        jax.ShapeDtypeStruct((1, 1), jnp.float32),
        jax.ShapeDtypeStruct((1, 128), jnp.float32),
    ],
)


def kernel(vp, vt, x0, x1, segment_ids):
    d2, c2, col, rpart = _tok_call(vp, vt, x0, x1)
    ids = segment_ids.astype(jnp.int32)
    sd, sc, cnt = _make_seg_call()(ids, d2.reshape(N), c2.reshape(N))
    loss11, dp = _fin_call(sd, sc, cnt, col, rpart)
    return loss11[0, 0], dp.reshape(D)


# speedup vs baseline: 3.6532x; 1.0584x over previous
import functools

import jax
import jax.numpy as jnp
from jax import lax
from jax.experimental import pallas as pl
from jax.experimental.pallas import tpu as pltpu
from jax.experimental.pallas import tpu_sc as plsc

N = 320000
D = 128
NSEG = 10000
NW = 32
CHUNK = N // NW
BT = 2560
RT = BT // 128
GRID = N // BT

_CSF = 0.5
_W = 0.1


def _tok_body(vp_ref, vt_ref, x0_ref, x1_ref, d_ref, c_ref, col_ref, r_ref):
    i = pl.program_id(0)

    def rs(x):
        return jnp.sum(x, axis=1).reshape(1, RT, 128)

    vp = vp_ref[...]
    vt = vt_ref[...]
    diff = vp - vt
    diff2 = diff * diff
    d2 = rs(diff2)
    num = rs(vp * vt)
    na = rs(vp * vp)
    nb = rs(vt * vt)
    r2 = num * lax.rsqrt(na * nb)
    cd = x0_ref[...] - x1_ref[...]
    c2 = jnp.sqrt(rs(cd * cd))

    d_ref[...] = d2
    c_ref[...] = c2

    col_part = jnp.sum(diff2, axis=0, keepdims=True)
    r_part = jnp.sum(r2.reshape(RT, 128), axis=0, keepdims=True)

    @pl.when(i == 0)
    def _init():
        col_ref[...] = col_part
        r_ref[...] = r_part

    @pl.when(i > 0)
    def _acc():
        col_ref[...] = col_ref[...] + col_part
        r_ref[...] = r_ref[...] + r_part


_tok_call = pl.pallas_call(
    _tok_body,
    grid=(GRID,),
    in_specs=[pl.BlockSpec((BT, 128), lambda i: (i, 0))] * 4,
    out_specs=[
        pl.BlockSpec((1, RT, 128), lambda i: (i, 0, 0)),
        pl.BlockSpec((1, RT, 128), lambda i: (i, 0, 0)),
        pl.BlockSpec((1, 128), lambda i: (0, 0)),
        pl.BlockSpec((1, 128), lambda i: (0, 0)),
    ],
    out_shape=[
        jax.ShapeDtypeStruct((GRID, RT, 128), jnp.float32),
        jax.ShapeDtypeStruct((GRID, RT, 128), jnp.float32),
        jax.ShapeDtypeStruct((1, 128), jnp.float32),
        jax.ShapeDtypeStruct((1, 128), jnp.float32),
    ],
    compiler_params=pltpu.CompilerParams(
        dimension_semantics=("arbitrary",),
    ),
)


@functools.lru_cache(maxsize=1)
def _make_seg_call():
    mesh = plsc.VectorSubcoreMesh(core_axis_name="c", subcore_axis_name="s")

    @functools.partial(
        pl.kernel,
        mesh=mesh,
        out_type=[
            jax.ShapeDtypeStruct((NW, NSEG), jnp.float32),
            jax.ShapeDtypeStruct((NW, NSEG), jnp.float32),
            jax.ShapeDtypeStruct((NW, NSEG), jnp.float32),
        ],
        compiler_params=pltpu.CompilerParams(needs_layout_passes=False),
        scratch_types=[
            pltpu.VMEM((CHUNK + 16,), jnp.int32),
            pltpu.VMEM((CHUNK,), jnp.float32),
            pltpu.VMEM((CHUNK,), jnp.float32),
            pltpu.VMEM((NSEG,), jnp.float32),
            pltpu.VMEM((NSEG,), jnp.float32),
            pltpu.VMEM((NSEG,), jnp.float32),
        ],
    )
    def seg_kernel(ids_hbm, d_hbm, c_hbm, sd_out, sc_out, cnt_out,
                   ids_v, d_v, c_v, acc_d, acc_c, acc_n):
        w = lax.axis_index("s") * 2 + lax.axis_index("c")
        base = w * CHUNK
        pltpu.sync_copy(ids_hbm.at[pl.ds(base, CHUNK)],
                        ids_v.at[pl.ds(8, CHUNK)])
        pltpu.sync_copy(d_hbm.at[pl.ds(base, CHUNK)], d_v)
        pltpu.sync_copy(c_hbm.at[pl.ds(base, CHUNK)], c_v)

        zf = jnp.zeros((16,), jnp.float32)

        def zero_body(j, carry):
            acc_d[pl.ds(j * 16, 16)] = zf
            acc_c[pl.ds(j * 16, 16)] = zf
            acc_n[pl.ds(j * 16, 16)] = zf
            return carry

        lax.fori_loop(0, NSEG // 16, zero_body, 0)

        lane = lax.iota(jnp.int32, 16)
        lanef = lane.astype(jnp.float32)

        def body(j, carry):
            off = 8 + j * 16
            ids = ids_v[pl.ds(off, 16)]
            prv = ids_v[pl.ds(off - 1, 16)]
            nxt = ids_v[pl.ds(off + 1, 16)]
            start = (ids != prv) | (lane == 0)
            end = (ids != nxt) | (lane == 15)
            dv = d_v[pl.ds(j * 16, 16)]
            cv = c_v[pl.ds(j * 16, 16)]
            cs_d = plsc.cumsum(dv)
            cs_c = plsc.cumsum(cv)
            plsc.addupdate_scatter(acc_d, [ids], cs_d, mask=end)
            plsc.addupdate_scatter(acc_d, [ids], dv - cs_d, mask=start)
            plsc.addupdate_scatter(acc_c, [ids], cs_c, mask=end)
            plsc.addupdate_scatter(acc_c, [ids], cv - cs_c, mask=start)
            plsc.addupdate_scatter(acc_n, [ids], lanef + 1.0, mask=end)
            plsc.addupdate_scatter(acc_n, [ids], -lanef, mask=start)
            return carry

        lax.fori_loop(0, CHUNK // 16, body, 0)

        pltpu.sync_copy(acc_d, sd_out.at[w])
        pltpu.sync_copy(acc_c, sc_out.at[w])
        pltpu.sync_copy(acc_n, cnt_out.at[w])

    return seg_kernel


def _fin_body(sd_ref, sc_ref, cnt_ref, col_ref, r_ref, loss_ref, dp_ref):
    sd = jnp.sum(sd_ref[...], axis=0)
    sc = jnp.sum(sc_ref[...], axis=0)
    cnt = jnp.sum(cnt_ref[...], axis=0)
    safe = jnp.maximum(cnt, 1.0)
    rs = jnp.sum(r_ref[...])
    cs = 1.0 - rs / N
    loss_seg = ((1.0 - _CSF) * 0.5 * sd + _CSF * cs * cnt) / safe
    loss_seg = loss_seg * jnp.exp(-_W * sc / safe)
    loss_ref[...] = (jnp.sum(loss_seg) / NSEG).reshape(1, 1)
    dp_ref[...] = col_ref[...] * (0.5 / N)


_fin_call = pl.pallas_call(
    _fin_body,
    out_shape=[
        jax.ShapeDtypeStruct((1, 1), jnp.float32),
        jax.ShapeDtypeStruct((1, 128), jnp.float32),
    ],
)


def kernel(vp, vt, x0, x1, segment_ids):
    d2, c2, col, rpart = _tok_call(vp, vt, x0, x1)
    ids = segment_ids.astype(jnp.int32)
    sd, sc, cnt = _make_seg_call()(ids, d2.reshape(N), c2.reshape(N))
    loss11, dp = _fin_call(sd, sc, cnt, col, rpart)
    return loss11[0, 0], dp.reshape(D)


# speedup vs baseline: 4.1233x; 1.1287x over previous
import functools

import jax
import jax.numpy as jnp
from jax import lax
from jax.experimental import pallas as pl
from jax.experimental.pallas import tpu as pltpu
from jax.experimental.pallas import tpu_sc as plsc

N = 320000
D = 128
NSEG = 10000
NW = 32
CHUNK = N // NW
BT = 6400
RT = BT // 128
GRID = N // BT

_CSF = 0.5
_W = 0.1


def _tok_body(vp_ref, vt_ref, x0_ref, x1_ref, d_ref, c_ref, col_ref, r_ref):
    i = pl.program_id(0)

    def rs(x):
        return jnp.sum(x, axis=1).reshape(1, RT, 128)

    vp = vp_ref[...]
    vt = vt_ref[...]
    diff = vp - vt
    diff2 = diff * diff
    d2 = rs(diff2)
    num = rs(vp * vt)
    na = rs(vp * vp)
    nb = rs(vt * vt)
    r2 = num * lax.rsqrt(na * nb)
    cd = x0_ref[...] - x1_ref[...]
    c2 = jnp.sqrt(rs(cd * cd))

    d_ref[...] = d2
    c_ref[...] = c2

    col_part = jnp.sum(diff2, axis=0, keepdims=True)
    r_part = jnp.sum(r2.reshape(RT, 128), axis=0, keepdims=True)

    @pl.when(i == 0)
    def _init():
        col_ref[...] = col_part
        r_ref[...] = r_part

    @pl.when(i > 0)
    def _acc():
        col_ref[...] = col_ref[...] + col_part
        r_ref[...] = r_ref[...] + r_part


_tok_call = pl.pallas_call(
    _tok_body,
    grid=(GRID,),
    in_specs=[pl.BlockSpec((BT, 128), lambda i: (i, 0))] * 4,
    out_specs=[
        pl.BlockSpec((1, RT, 128), lambda i: (i, 0, 0)),
        pl.BlockSpec((1, RT, 128), lambda i: (i, 0, 0)),
        pl.BlockSpec((1, 128), lambda i: (0, 0)),
        pl.BlockSpec((1, 128), lambda i: (0, 0)),
    ],
    out_shape=[
        jax.ShapeDtypeStruct((GRID, RT, 128), jnp.float32),
        jax.ShapeDtypeStruct((GRID, RT, 128), jnp.float32),
        jax.ShapeDtypeStruct((1, 128), jnp.float32),
        jax.ShapeDtypeStruct((1, 128), jnp.float32),
    ],
    compiler_params=pltpu.CompilerParams(
        dimension_semantics=("arbitrary",),
    ),
)


@functools.lru_cache(maxsize=1)
def _make_seg_call():
    mesh = plsc.VectorSubcoreMesh(core_axis_name="c", subcore_axis_name="s")

    @functools.partial(
        pl.kernel,
        mesh=mesh,
        out_type=[
            jax.ShapeDtypeStruct((NW, NSEG), jnp.float32),
            jax.ShapeDtypeStruct((NW, NSEG), jnp.float32),
            jax.ShapeDtypeStruct((NW, NSEG), jnp.float32),
        ],
        compiler_params=pltpu.CompilerParams(needs_layout_passes=False),
        scratch_types=[
            pltpu.VMEM((CHUNK + 16,), jnp.int32),
            pltpu.VMEM((CHUNK,), jnp.float32),
            pltpu.VMEM((CHUNK,), jnp.float32),
            pltpu.VMEM((NSEG,), jnp.float32),
            pltpu.VMEM((NSEG,), jnp.float32),
            pltpu.VMEM((NSEG,), jnp.float32),
        ],
    )
    def seg_kernel(ids_hbm, d_hbm, c_hbm, sd_out, sc_out, cnt_out,
                   ids_v, d_v, c_v, acc_d, acc_c, acc_n):
        w = lax.axis_index("s") * 2 + lax.axis_index("c")
        base = w * CHUNK
        pltpu.sync_copy(ids_hbm.at[pl.ds(base, CHUNK)],
                        ids_v.at[pl.ds(8, CHUNK)])
        pltpu.sync_copy(d_hbm.at[pl.ds(base, CHUNK)], d_v)
        pltpu.sync_copy(c_hbm.at[pl.ds(base, CHUNK)], c_v)

        zf = jnp.zeros((16,), jnp.float32)

        def zero_body(j, carry):
            acc_d[pl.ds(j * 16, 16)] = zf
            acc_c[pl.ds(j * 16, 16)] = zf
            acc_n[pl.ds(j * 16, 16)] = zf
            return carry

        lax.fori_loop(0, NSEG // 16, zero_body, 0)

        lane = lax.iota(jnp.int32, 16)
        lanef = lane.astype(jnp.float32)

        def body(j, carry):
            off = 8 + j * 16
            ids = ids_v[pl.ds(off, 16)]
            prv = ids_v[pl.ds(off - 1, 16)]
            nxt = ids_v[pl.ds(off + 1, 16)]
            start = (ids != prv) | (lane == 0)
            end = (ids != nxt) | (lane == 15)
            dv = d_v[pl.ds(j * 16, 16)]
            cv = c_v[pl.ds(j * 16, 16)]
            cs_d = plsc.cumsum(dv)
            cs_c = plsc.cumsum(cv)
            plsc.addupdate_scatter(acc_d, [ids], cs_d, mask=end)
            plsc.addupdate_scatter(acc_d, [ids], dv - cs_d, mask=start)
            plsc.addupdate_scatter(acc_c, [ids], cs_c, mask=end)
            plsc.addupdate_scatter(acc_c, [ids], cv - cs_c, mask=start)
            plsc.addupdate_scatter(acc_n, [ids], lanef + 1.0, mask=end)
            plsc.addupdate_scatter(acc_n, [ids], -lanef, mask=start)
            return carry

        lax.fori_loop(0, CHUNK // 16, body, 0)

        pltpu.sync_copy(acc_d, sd_out.at[w])
        pltpu.sync_copy(acc_c, sc_out.at[w])
        pltpu.sync_copy(acc_n, cnt_out.at[w])

    return seg_kernel


def _fin_body(sd_ref, sc_ref, cnt_ref, col_ref, r_ref, loss_ref, dp_ref):
    sd = jnp.sum(sd_ref[...], axis=0)
    sc = jnp.sum(sc_ref[...], axis=0)
    cnt = jnp.sum(cnt_ref[...], axis=0)
    safe = jnp.maximum(cnt, 1.0)
    rs = jnp.sum(r_ref[...])
    cs = 1.0 - rs / N
    loss_seg = ((1.0 - _CSF) * 0.5 * sd + _CSF * cs * cnt) / safe
    loss_seg = loss_seg * jnp.exp(-_W * sc / safe)
    loss_ref[...] = (jnp.sum(loss_seg) / NSEG).reshape(1, 1)
    dp_ref[...] = col_ref[...] * (0.5 / N)


_fin_call = pl.pallas_call(
    _fin_body,
    out_shape=[
        jax.ShapeDtypeStruct((1, 1), jnp.float32),
        jax.ShapeDtypeStruct((1, 128), jnp.float32),
    ],
)


def kernel(vp, vt, x0, x1, segment_ids):
    d2, c2, col, rpart = _tok_call(vp, vt, x0, x1)
    ids = segment_ids.astype(jnp.int32)
    sd, sc, cnt = _make_seg_call()(ids, d2.reshape(N), c2.reshape(N))
    loss11, dp = _fin_call(sd, sc, cnt, col, rpart)
    return loss11[0, 0], dp.reshape(D)


# speedup vs baseline: 4.3192x; 1.0475x over previous
import functools

import jax
import jax.numpy as jnp
from jax import lax
from jax.experimental import pallas as pl
from jax.experimental.pallas import tpu as pltpu
from jax.experimental.pallas import tpu_sc as plsc

N = 320000
D = 128
NSEG = 10000
NW = 32
CHUNK = N // NW
BT = 6400
RT = BT // 128
GRID = N // BT

_CSF = 0.5
_W = 0.1


def _tok_body(vp_ref, vt_ref, x0_ref, x1_ref, d_ref, c_ref, col_ref, r_ref):
    i = pl.program_id(0)

    def rs(x):
        return jnp.sum(x, axis=1).reshape(1, RT, 128)

    vp = vp_ref[...]
    vt = vt_ref[...]
    diff = vp - vt
    diff2 = diff * diff
    num = rs(vp * vt)
    na = rs(vp * vp)
    nb = rs(vt * vt)
    d2 = na + nb - 2.0 * num
    r2 = num * lax.rsqrt(na * nb)
    cd = x0_ref[...] - x1_ref[...]
    c2 = jnp.sqrt(rs(cd * cd))

    d_ref[...] = d2
    c_ref[...] = c2

    col_part = jnp.sum(diff2, axis=0, keepdims=True)
    r_part = jnp.sum(r2.reshape(RT, 128), axis=0, keepdims=True)

    @pl.when(i == 0)
    def _init():
        col_ref[...] = col_part
        r_ref[...] = r_part

    @pl.when(i > 0)
    def _acc():
        col_ref[...] = col_ref[...] + col_part
        r_ref[...] = r_ref[...] + r_part


_tok_call = pl.pallas_call(
    _tok_body,
    grid=(GRID,),
    in_specs=[pl.BlockSpec((BT, 128), lambda i: (i, 0))] * 4,
    out_specs=[
        pl.BlockSpec((1, RT, 128), lambda i: (i, 0, 0)),
        pl.BlockSpec((1, RT, 128), lambda i: (i, 0, 0)),
        pl.BlockSpec((1, 128), lambda i: (0, 0)),
        pl.BlockSpec((1, 128), lambda i: (0, 0)),
    ],
    out_shape=[
        jax.ShapeDtypeStruct((GRID, RT, 128), jnp.float32),
        jax.ShapeDtypeStruct((GRID, RT, 128), jnp.float32),
        jax.ShapeDtypeStruct((1, 128), jnp.float32),
        jax.ShapeDtypeStruct((1, 128), jnp.float32),
    ],
    compiler_params=pltpu.CompilerParams(
        dimension_semantics=("arbitrary",),
    ),
)


@functools.lru_cache(maxsize=1)
def _make_seg_call():
    mesh = plsc.VectorSubcoreMesh(core_axis_name="c", subcore_axis_name="s")

    @functools.partial(
        pl.kernel,
        mesh=mesh,
        out_type=[
            jax.ShapeDtypeStruct((NW, NSEG), jnp.float32),
            jax.ShapeDtypeStruct((NW, NSEG), jnp.float32),
            jax.ShapeDtypeStruct((NW, NSEG), jnp.float32),
        ],
        compiler_params=pltpu.CompilerParams(needs_layout_passes=False),
        scratch_types=[
            pltpu.VMEM((CHUNK + 16,), jnp.int32),
            pltpu.VMEM((CHUNK,), jnp.float32),
            pltpu.VMEM((CHUNK,), jnp.float32),
            pltpu.VMEM((NSEG,), jnp.float32),
            pltpu.VMEM((NSEG,), jnp.float32),
            pltpu.VMEM((NSEG,), jnp.float32),
        ],
    )
    def seg_kernel(ids_hbm, d_hbm, c_hbm, sd_out, sc_out, cnt_out,
                   ids_v, d_v, c_v, acc_d, acc_c, acc_n):
        w = lax.axis_index("s") * 2 + lax.axis_index("c")
        base = w * CHUNK
        pltpu.sync_copy(ids_hbm.at[pl.ds(base, CHUNK)],
                        ids_v.at[pl.ds(8, CHUNK)])
        pltpu.sync_copy(d_hbm.at[pl.ds(base, CHUNK)], d_v)
        pltpu.sync_copy(c_hbm.at[pl.ds(base, CHUNK)], c_v)

        zf = jnp.zeros((16,), jnp.float32)

        def zero_body(j, carry):
            acc_d[pl.ds(j * 16, 16)] = zf
            acc_c[pl.ds(j * 16, 16)] = zf
            acc_n[pl.ds(j * 16, 16)] = zf
            return carry

        lax.fori_loop(0, NSEG // 16, zero_body, 0)

        lane = lax.iota(jnp.int32, 16)
        lanef = lane.astype(jnp.float32)

        def body(j, carry):
            off = 8 + j * 16
            ids = ids_v[pl.ds(off, 16)]
            prv = ids_v[pl.ds(off - 1, 16)]
            nxt = ids_v[pl.ds(off + 1, 16)]
            start = (ids != prv) | (lane == 0)
            end = (ids != nxt) | (lane == 15)
            dv = d_v[pl.ds(j * 16, 16)]
            cv = c_v[pl.ds(j * 16, 16)]
            cs_d = plsc.cumsum(dv)
            cs_c = plsc.cumsum(cv)
            plsc.addupdate_scatter(acc_d, [ids], cs_d, mask=end)
            plsc.addupdate_scatter(acc_d, [ids], dv - cs_d, mask=start)
            plsc.addupdate_scatter(acc_c, [ids], cs_c, mask=end)
            plsc.addupdate_scatter(acc_c, [ids], cv - cs_c, mask=start)
            plsc.addupdate_scatter(acc_n, [ids], lanef + 1.0, mask=end)
            plsc.addupdate_scatter(acc_n, [ids], -lanef, mask=start)
            return carry

        lax.fori_loop(0, CHUNK // 16, body, 0)

        pltpu.sync_copy(acc_d, sd_out.at[w])
        pltpu.sync_copy(acc_c, sc_out.at[w])
        pltpu.sync_copy(acc_n, cnt_out.at[w])

    return seg_kernel


def _fin_body(sd_ref, sc_ref, cnt_ref, col_ref, r_ref, loss_ref, dp_ref):
    sd = jnp.sum(sd_ref[...], axis=0)
    sc = jnp.sum(sc_ref[...], axis=0)
    cnt = jnp.sum(cnt_ref[...], axis=0)
    safe = jnp.maximum(cnt, 1.0)
    rs = jnp.sum(r_ref[...])
    cs = 1.0 - rs / N
    loss_seg = ((1.0 - _CSF) * 0.5 * sd + _CSF * cs * cnt) / safe
    loss_seg = loss_seg * jnp.exp(-_W * sc / safe)
    loss_ref[...] = (jnp.sum(loss_seg) / NSEG).reshape(1, 1)
    dp_ref[...] = col_ref[...] * (0.5 / N)


_fin_call = pl.pallas_call(
    _fin_body,
    out_shape=[
        jax.ShapeDtypeStruct((1, 1), jnp.float32),
        jax.ShapeDtypeStruct((1, 128), jnp.float32),
    ],
)


def kernel(vp, vt, x0, x1, segment_ids):
    d2, c2, col, rpart = _tok_call(vp, vt, x0, x1)
    ids = segment_ids.astype(jnp.int32)
    sd, sc, cnt = _make_seg_call()(ids, d2.reshape(N), c2.reshape(N))
    loss11, dp = _fin_call(sd, sc, cnt, col, rpart)
    return loss11[0, 0], dp.reshape(D)
